# baseline (speedup 1.0000x reference)
import dataclasses
import functools

import jax
import jax.numpy as jnp
from jax import lax
from jax.experimental import pallas as pl
from jax.experimental.pallas import tpu as pltpu
from jax.experimental.pallas import tpu_sc as plsc

N = 10000
E = 320000
D = 128

NC = 2
NS = 16
NW = NC * NS
KE = 80
NCHP = 125
EPW = NCHP * KE
WPB = 8
FW = NCHP // WPB
TWC = NCHP % WPB
NBUF = 2
RPS = N // NS

_mesh = plsc.VectorSubcoreMesh(
    core_axis_name="c", subcore_axis_name="s", num_cores=NC, num_subcores=NS
)


def _worker_id():
    return lax.axis_index("s") * NC + lax.axis_index("c")


def _fill(buf, nrows, value):
    @pl.loop(0, nrows)
    def _(r):
        @pl.loop(0, D // 16)
        def _(cc):
            buf[r, pl.ds(cc * 16, 16)] = jnp.full((16,), value, jnp.float32)


def _zero_acc_slice(zbuf, acc_sh, sid):
    @pl.loop(0, RPS // KE)
    def _(j):
        pltpu.sync_copy(zbuf, acc_sh.at[pl.ds(sid * RPS + j * KE, KE)])

    rem = RPS - (RPS // KE) * KE
    if rem:
        pltpu.sync_copy(
            zbuf.at[pl.ds(0, rem)],
            acc_sh.at[pl.ds(sid * RPS + (RPS // KE) * KE, rem)],
        )


HRD = 10240

_cp = pltpu.CompilerParams()
if "needs_layout_passes" in pltpu.CompilerParams.__dataclass_fields__:
    _cp = dataclasses.replace(_cp, needs_layout_passes=False)


@functools.partial(
    pl.kernel,
    out_type=jax.ShapeDtypeStruct((NW * HRD,), jnp.float32),
    mesh=_mesh,
    compiler_params=_cp,
    scratch_types=[
        pltpu.VMEM((NCHP, KE), jnp.int32),
        pltpu.VMEM((HRD,), jnp.float32),
        pltpu.SemaphoreType.DMA,
    ],
)
def _hist_kernel(dst_hbm, out_hbm, dst_v, lh_v, sem):
    wid = _worker_id()

    pltpu.async_copy(dst_hbm.at[wid], dst_v, sem)

    @pl.loop(0, HRD // 16)
    def _(k):
        lh_v[pl.ds(k * 16, 16)] = jnp.zeros((16,), jnp.float32)

    pltpu.make_async_copy(dst_hbm.at[wid], dst_v, sem).wait()

    ones16 = jnp.full((16,), 1.0, jnp.float32)

    @pl.loop(0, NCHP)
    def _(i):
        @pl.loop(0, KE // 16)
        def _(c):
            idx = dst_v[i, pl.ds(c * 16, 16)]
            plsc.addupdate_scatter(lh_v, [idx], ones16)

    pltpu.sync_copy(lh_v, out_hbm.at[pl.ds(wid * HRD, HRD)])


@functools.partial(
    pl.kernel,
    out_type=jax.ShapeDtypeStruct((NC, NS, RPS, D), jnp.float32),
    mesh=_mesh,
    scratch_types=[
        pltpu.VMEM((NCHP, KE), jnp.int32),
        pltpu.VMEM((WPB, KE), jnp.int32),
        pltpu.VMEM((WPB, KE), jnp.int32),
        pltpu.VMEM((KE, D), jnp.float32),
        pltpu.VMEM((KE, D), jnp.float32),
        pltpu.VMEM_SHARED((N, D), jnp.float32),
        pltpu.SemaphoreType.DMA,
        pltpu.SemaphoreType.DMA,
        pltpu.SemaphoreType.DMA,
        pltpu.SemaphoreType.DMA,
    ],
)
def _agg_kernel(h_hbm, src_hbm, dst_hbm, out_hbm,
                src_v, dwa, dwb, rows0, rows1, acc_sh,
                gs0, gs1, wsa, wsb):
    cid = lax.axis_index("c")
    sid = lax.axis_index("s")
    wid = _worker_id()

    _fill(rows0, KE, 0.0)
    _zero_acc_slice(rows0, acc_sh, sid)

    pltpu.sync_copy(src_hbm.at[wid], src_v)
    pltpu.sync_copy(dst_hbm.at[wid, pl.ds(0, WPB)], dwa)
    pltpu.async_copy(dst_hbm.at[wid, pl.ds(WPB, WPB)], dwb, wsb)
    plsc.subcore_barrier()

    bufs = (rows0, rows1)
    sems = (gs0, gs1)

    def _gather(i, buf, sem):
        pltpu.async_copy(h_hbm.at[src_v.at[i]], buf, sem)

    def _gwait(i, buf, sem):
        pltpu.make_async_copy(h_hbm.at[src_v.at[i]], buf, sem).wait()

    def _win_start(w, nch, buf, sem):
        pltpu.async_copy(
            dst_hbm.at[wid, pl.ds(w * WPB, nch)], buf.at[pl.ds(0, nch)], sem
        )

    def _win_wait(w, nch, buf, sem):
        pltpu.make_async_copy(
            dst_hbm.at[wid, pl.ds(w * WPB, nch)], buf.at[pl.ds(0, nch)], sem
        ).wait()

    def _chunk_step(i, j, win_buf):
        buf, sem = bufs[j % NBUF], sems[j % NBUF]
        _gwait(i, buf, sem)
        pltpu.sync_copy(buf, acc_sh.at[win_buf.at[j]], add=True)

        @pl.when(i + NBUF < NCHP)
        def _():
            _gather(i + NBUF, buf, sem)

    for b in range(NBUF):
        _gather(b, bufs[b], sems[b])

    @pl.loop(0, FW - 1, step=2)
    def _(w):
        base = w * WPB
        for j in range(WPB):
            _chunk_step(base + j, j, dwa)

        @pl.when(w + 2 < FW)
        def _():
            _win_start(w + 2, WPB, dwa, wsa)

        _win_wait(w + 1, WPB, dwb, wsb)
        for j in range(WPB):
            _chunk_step(base + WPB + j, j, dwb)

        @pl.when(w + 3 < FW)
        def _():
            _win_start(w + 3, WPB, dwb, wsb)

        @pl.when(w + 2 < FW)
        def _():
            _win_wait(w + 2, WPB, dwa, wsa)

    _win_start(FW, TWC, dwb, wsb)
    for j in range(WPB):
        _chunk_step((FW - 1) * WPB + j, j, dwa)

    _win_wait(FW, TWC, dwb, wsb)
    for j in range(TWC):
        _chunk_step(FW * WPB + j, j, dwb)

    plsc.subcore_barrier()
    pltpu.sync_copy(
        acc_sh.at[pl.ds(sid * RPS, RPS)],
        out_hbm.at[cid, sid],
    )


_RB = 400
_GRID = N // _RB

_row_spec = pl.BlockSpec((_RB, D), lambda i: (i, 0))
_pair_spec = pl.BlockSpec((NC, _RB, D), lambda i: (0, i, 0))
_histp_spec = pl.BlockSpec((NW, _RB, 1), lambda i: (0, i, 0))
_w_spec = pl.BlockSpec((D, D), lambda i: (0, 0))
_b_spec = pl.BlockSpec((1, D), lambda i: (0, 0))


def _dinv_of(hist_ref):
    deg = jnp.sum(hist_ref[...], axis=0)[:, 0] + 1.0
    return lax.rsqrt(deg)


def _prep_body(x_ref, w_ref, hist_ref, o_ref):
    dinv = _dinv_of(hist_ref)
    h = jnp.dot(x_ref[...], w_ref[...], preferred_element_type=jnp.float32)
    o_ref[...] = h * dinv[:, None]


_prep = pl.pallas_call(
    _prep_body,
    grid=(_GRID,),
    in_specs=[_row_spec, _w_spec, _histp_spec],
    out_specs=_row_spec,
    out_shape=jax.ShapeDtypeStruct((N, D), jnp.float32),
)


def _mid_body(p_ref, hp_ref, b_ref, w_ref, hist_ref, o_ref):
    dinv = _dinv_of(hist_ref)
    pre = dinv[:, None] * (p_ref[0] + p_ref[1] + hp_ref[...]) + b_ref[...]
    act = jnp.maximum(pre, 0.0)
    h = jnp.dot(act, w_ref[...], preferred_element_type=jnp.float32)
    o_ref[...] = h * dinv[:, None]


_mid = pl.pallas_call(
    _mid_body,
    grid=(_GRID,),
    in_specs=[_pair_spec, _row_spec, _b_spec, _w_spec, _histp_spec],
    out_specs=_row_spec,
    out_shape=jax.ShapeDtypeStruct((N, D), jnp.float32),
)


def _final_body(p_ref, hp_ref, b_ref, hist_ref, o_ref):
    dinv = _dinv_of(hist_ref)
    o_ref[...] = (
        dinv[:, None] * (p_ref[0] + p_ref[1] + hp_ref[...]) + b_ref[...]
    )


_final = pl.pallas_call(
    _final_body,
    grid=(_GRID,),
    in_specs=[_pair_spec, _row_spec, _b_spec, _histp_spec],
    out_specs=_row_spec,
    out_shape=jax.ShapeDtypeStruct((N, D), jnp.float32),
)


def kernel(x, edge_index, W1, b1, W2, b2):
    src = edge_index[0].astype(jnp.int32).reshape(NW, NCHP, KE)
    dst = edge_index[1].astype(jnp.int32).reshape(NW, NCHP, KE)
    b1r = b1.reshape(1, D)
    b2r = b2.reshape(1, D)

    hist = _hist_kernel(dst).reshape(NW, HRD)[:, :N].reshape(NW, N, 1)

    h1p = _prep(x, W1, hist)
    p = _agg_kernel(h1p, src, dst).reshape(NC, N, D)
    h2p = _mid(p, h1p, b1r, W2, hist)
    q = _agg_kernel(h2p, src, dst).reshape(NC, N, D)
    out = _final(q, h2p, b2r, hist)
    return out


# speedup vs baseline: 1.5718x; 1.5718x over previous
import dataclasses
import functools

import jax
import jax.numpy as jnp
from jax import lax
from jax.experimental import pallas as pl
from jax.experimental.pallas import tpu as pltpu
from jax.experimental.pallas import tpu_sc as plsc

N = 10000
E = 320000
D = 128

NC = 2
NS = 16
NW = NC * NS
KE = 80
NCHP = 125
EPW = NCHP * KE
WPB = 8
FW = NCHP // WPB
TWC = NCHP % WPB
NBUF = 2
RPS = N // NS

_mesh = plsc.VectorSubcoreMesh(
    core_axis_name="c", subcore_axis_name="s", num_cores=NC, num_subcores=NS
)


def _worker_id():
    return lax.axis_index("s") * NC + lax.axis_index("c")


def _fill(buf, nrows, value):
    @pl.loop(0, nrows)
    def _(r):
        @pl.loop(0, D // 16)
        def _(cc):
            buf[r, pl.ds(cc * 16, 16)] = jnp.full((16,), value, jnp.float32)


def _zero_acc_slice(zbuf, acc_sh, sid):
    @pl.loop(0, RPS // KE)
    def _(j):
        pltpu.sync_copy(zbuf, acc_sh.at[pl.ds(sid * RPS + j * KE, KE)])

    rem = RPS - (RPS // KE) * KE
    if rem:
        pltpu.sync_copy(
            zbuf.at[pl.ds(0, rem)],
            acc_sh.at[pl.ds(sid * RPS + (RPS // KE) * KE, rem)],
        )


HRD = 10240

_cp = pltpu.CompilerParams()
if "needs_layout_passes" in pltpu.CompilerParams.__dataclass_fields__:
    _cp = dataclasses.replace(_cp, needs_layout_passes=False)


@functools.partial(
    pl.kernel,
    out_type=jax.ShapeDtypeStruct((NW * HRD,), jnp.float32),
    mesh=_mesh,
    compiler_params=_cp,
    scratch_types=[
        pltpu.VMEM((NCHP, KE), jnp.int32),
        pltpu.VMEM((HRD,), jnp.float32),
        pltpu.SemaphoreType.DMA,
    ],
)
def _hist_kernel(dst_hbm, out_hbm, dst_v, lh_v, sem):
    wid = _worker_id()

    pltpu.async_copy(dst_hbm.at[wid], dst_v, sem)

    @pl.loop(0, HRD // 16)
    def _(k):
        lh_v[pl.ds(k * 16, 16)] = jnp.zeros((16,), jnp.float32)

    pltpu.make_async_copy(dst_hbm.at[wid], dst_v, sem).wait()

    ones16 = jnp.full((16,), 1.0, jnp.float32)

    @pl.loop(0, NCHP)
    def _(i):
        @pl.loop(0, KE // 16)
        def _(c):
            idx = dst_v[i, pl.ds(c * 16, 16)]
            plsc.addupdate_scatter(lh_v, [idx], ones16)

    pltpu.sync_copy(lh_v, out_hbm.at[pl.ds(wid * HRD, HRD)])


@functools.partial(
    pl.kernel,
    out_type=jax.ShapeDtypeStruct((NC, NS, RPS, D), jnp.float32),
    mesh=_mesh,
    scratch_types=[
        pltpu.VMEM((NCHP, KE), jnp.int32),
        pltpu.VMEM((WPB, KE), jnp.int32),
        pltpu.VMEM((WPB, KE), jnp.int32),
        pltpu.VMEM((KE, D), jnp.float32),
        pltpu.VMEM((KE, D), jnp.float32),
        pltpu.VMEM_SHARED((N, D), jnp.float32),
        pltpu.SemaphoreType.DMA,
        pltpu.SemaphoreType.DMA,
        pltpu.SemaphoreType.DMA,
        pltpu.SemaphoreType.DMA,
    ],
)
def _agg_kernel(h_hbm, src_hbm, dst_hbm, out_hbm,
                src_v, dwa, dwb, rows0, rows1, acc_sh,
                gs0, gs1, wsa, wsb):
    cid = lax.axis_index("c")
    sid = lax.axis_index("s")
    wid = _worker_id()

    _fill(rows0, KE, 0.0)
    _zero_acc_slice(rows0, acc_sh, sid)

    pltpu.sync_copy(src_hbm.at[wid], src_v)
    pltpu.sync_copy(dst_hbm.at[wid, pl.ds(0, WPB)], dwa)
    pltpu.async_copy(dst_hbm.at[wid, pl.ds(WPB, WPB)], dwb, wsb)
    plsc.subcore_barrier()

    bufs = (rows0, rows1)
    sems = (gs0, gs1)

    def _gather(i, buf, sem):
        pltpu.async_copy(h_hbm.at[src_v.at[i]], buf, sem)

    def _gwait(i, buf, sem):
        pltpu.make_async_copy(h_hbm.at[src_v.at[i]], buf, sem).wait()

    def _win_start(w, nch, buf, sem):
        pltpu.async_copy(
            dst_hbm.at[wid, pl.ds(w * WPB, nch)], buf.at[pl.ds(0, nch)], sem
        )

    def _win_wait(w, nch, buf, sem):
        pltpu.make_async_copy(
            dst_hbm.at[wid, pl.ds(w * WPB, nch)], buf.at[pl.ds(0, nch)], sem
        ).wait()

    def _chunk_step(i, j, win_buf):
        buf, sem = bufs[j % NBUF], sems[j % NBUF]
        _gwait(i, buf, sem)
        pltpu.sync_copy(buf, acc_sh.at[win_buf.at[j]], add=True)

        @pl.when(i + NBUF < NCHP)
        def _():
            _gather(i + NBUF, buf, sem)

    for b in range(NBUF):
        _gather(b, bufs[b], sems[b])

    @pl.loop(0, FW - 1, step=2)
    def _(w):
        base = w * WPB
        for j in range(WPB):
            _chunk_step(base + j, j, dwa)

        @pl.when(w + 2 < FW)
        def _():
            _win_start(w + 2, WPB, dwa, wsa)

        _win_wait(w + 1, WPB, dwb, wsb)
        for j in range(WPB):
            _chunk_step(base + WPB + j, j, dwb)

        @pl.when(w + 3 < FW)
        def _():
            _win_start(w + 3, WPB, dwb, wsb)

        @pl.when(w + 2 < FW)
        def _():
            _win_wait(w + 2, WPB, dwa, wsa)

    _win_start(FW, TWC, dwb, wsb)
    for j in range(WPB):
        _chunk_step((FW - 1) * WPB + j, j, dwa)

    _win_wait(FW, TWC, dwb, wsb)
    for j in range(TWC):
        _chunk_step(FW * WPB + j, j, dwb)

    plsc.subcore_barrier()
    pltpu.sync_copy(
        acc_sh.at[pl.ds(sid * RPS, RPS)],
        out_hbm.at[cid, sid],
    )


_RB = 400
_GRID = N // _RB

_row_spec = pl.BlockSpec((_RB, D), lambda i: (i, 0))
_pair_spec = pl.BlockSpec((NC, _RB, D), lambda i: (0, i, 0))
_w_spec = pl.BlockSpec((D, D), lambda i: (0, 0))
_b_spec = pl.BlockSpec((1, D), lambda i: (0, 0))


def _prep_body(part_ref, x_ref, w_ref, hp_ref, db_ref):
    deg_flat = jnp.sum(part_ref[...], axis=0) + 1.0
    dinv_flat = lax.rsqrt(deg_flat)
    eye = (
        lax.broadcasted_iota(jnp.int32, (D, D), 0)
        == lax.broadcasted_iota(jnp.int32, (D, D), 1)
    ).astype(jnp.float32)
    h = jnp.dot(x_ref[...], w_ref[...], preferred_element_type=jnp.float32)
    for k in range((N + D - 1) // D):
        sz = min(D, N - k * D)
        row = dinv_flat[k : k + 1, :]
        col = jnp.sum(
            jnp.broadcast_to(row, (D, D)) * eye, axis=1, keepdims=True
        )
        blk = jnp.broadcast_to(col, (D, D))[:sz]
        db_ref[pl.ds(k * D, sz), :] = blk
        hp_ref[pl.ds(k * D, sz), :] = h[k * D : k * D + sz] * blk


_prep = pl.pallas_call(
    _prep_body,
    out_shape=[
        jax.ShapeDtypeStruct((N, D), jnp.float32),
        jax.ShapeDtypeStruct((N, D), jnp.float32),
    ],
)


def _mid_body(p_ref, hp_ref, b_ref, w_ref, db_ref, o_ref):
    dinv = db_ref[...]
    pre = dinv * (p_ref[0] + p_ref[1] + hp_ref[...]) + b_ref[...]
    act = jnp.maximum(pre, 0.0)
    h = jnp.dot(act, w_ref[...], preferred_element_type=jnp.float32)
    o_ref[...] = h * dinv


_mid = pl.pallas_call(
    _mid_body,
    grid=(_GRID,),
    in_specs=[_pair_spec, _row_spec, _b_spec, _w_spec, _row_spec],
    out_specs=_row_spec,
    out_shape=jax.ShapeDtypeStruct((N, D), jnp.float32),
)


def _final_body(p_ref, hp_ref, b_ref, db_ref, o_ref):
    o_ref[...] = (
        db_ref[...] * (p_ref[0] + p_ref[1] + hp_ref[...]) + b_ref[...]
    )


_final = pl.pallas_call(
    _final_body,
    grid=(_GRID,),
    in_specs=[_pair_spec, _row_spec, _b_spec, _row_spec],
    out_specs=_row_spec,
    out_shape=jax.ShapeDtypeStruct((N, D), jnp.float32),
)


def kernel(x, edge_index, W1, b1, W2, b2):
    src = edge_index[0].astype(jnp.int32).reshape(NW, NCHP, KE)
    dst = edge_index[1].astype(jnp.int32).reshape(NW, NCHP, KE)
    b1r = b1.reshape(1, D)
    b2r = b2.reshape(1, D)

    hist = _hist_kernel(dst).reshape(NW, HRD // D, D)

    h1p, dinvb = _prep(hist, x, W1)
    p = _agg_kernel(h1p, src, dst).reshape(NC, N, D)
    h2p = _mid(p, h1p, b1r, W2, dinvb)
    q = _agg_kernel(h2p, src, dst).reshape(NC, N, D)
    out = _final(q, h2p, b2r, dinvb)
    return out


# speedup vs baseline: 1.7910x; 1.1394x over previous
import dataclasses
import functools

import jax
import jax.numpy as jnp
from jax import lax
from jax.experimental import pallas as pl
from jax.experimental.pallas import tpu as pltpu
from jax.experimental.pallas import tpu_sc as plsc

N = 10000
E = 320000
D = 128

NC = 2
NS = 16
NW = NC * NS
KE = 80
NCHP = 125
EPW = NCHP * KE
WPB = 24
FW = NCHP // WPB
TWC = NCHP % WPB
NBUF = 3
RPS = N // NS

_mesh = plsc.VectorSubcoreMesh(
    core_axis_name="c", subcore_axis_name="s", num_cores=NC, num_subcores=NS
)


def _worker_id():
    return lax.axis_index("s") * NC + lax.axis_index("c")


def _fill(buf, nrows, value):
    @pl.loop(0, nrows)
    def _(r):
        @pl.loop(0, D // 16)
        def _(cc):
            buf[r, pl.ds(cc * 16, 16)] = jnp.full((16,), value, jnp.float32)


def _zero_acc_slice(zbuf, acc_sh, sid):
    @pl.loop(0, RPS // KE)
    def _(j):
        pltpu.sync_copy(zbuf, acc_sh.at[pl.ds(sid * RPS + j * KE, KE)])

    rem = RPS - (RPS // KE) * KE
    if rem:
        pltpu.sync_copy(
            zbuf.at[pl.ds(0, rem)],
            acc_sh.at[pl.ds(sid * RPS + (RPS // KE) * KE, rem)],
        )


HRD = 10240

_cp = pltpu.CompilerParams()
if "needs_layout_passes" in pltpu.CompilerParams.__dataclass_fields__:
    _cp = dataclasses.replace(_cp, needs_layout_passes=False)


@functools.partial(
    pl.kernel,
    out_type=jax.ShapeDtypeStruct((NW * HRD,), jnp.float32),
    mesh=_mesh,
    compiler_params=_cp,
    scratch_types=[
        pltpu.VMEM((NCHP, KE), jnp.int32),
        pltpu.VMEM((HRD,), jnp.float32),
        pltpu.SemaphoreType.DMA,
    ],
)
def _hist_kernel(dst_hbm, out_hbm, dst_v, lh_v, sem):
    wid = _worker_id()

    pltpu.async_copy(dst_hbm.at[wid], dst_v, sem)

    @pl.loop(0, HRD // 16)
    def _(k):
        lh_v[pl.ds(k * 16, 16)] = jnp.zeros((16,), jnp.float32)

    pltpu.make_async_copy(dst_hbm.at[wid], dst_v, sem).wait()

    ones16 = jnp.full((16,), 1.0, jnp.float32)

    @pl.loop(0, NCHP)
    def _(i):
        @pl.loop(0, KE // 16)
        def _(c):
            idx = dst_v[i, pl.ds(c * 16, 16)]
            plsc.addupdate_scatter(lh_v, [idx], ones16)

    pltpu.sync_copy(lh_v, out_hbm.at[pl.ds(wid * HRD, HRD)])


@functools.partial(
    pl.kernel,
    out_type=jax.ShapeDtypeStruct((NC, NS, RPS, D), jnp.float32),
    mesh=_mesh,
    scratch_types=[
        pltpu.VMEM((WPB, KE), jnp.int32),
        pltpu.VMEM((WPB, KE), jnp.int32),
        pltpu.VMEM((WPB, KE), jnp.int32),
        pltpu.VMEM((WPB, KE), jnp.int32),
        pltpu.VMEM((KE, D), jnp.float32),
        pltpu.VMEM((KE, D), jnp.float32),
        pltpu.VMEM((KE, D), jnp.float32),
        pltpu.VMEM_SHARED((N, D), jnp.float32),
        pltpu.SemaphoreType.DMA,
        pltpu.SemaphoreType.DMA,
        pltpu.SemaphoreType.DMA,
        pltpu.SemaphoreType.DMA,
        pltpu.SemaphoreType.DMA,
        pltpu.SemaphoreType.DMA,
        pltpu.SemaphoreType.DMA,
    ],
)
def _agg_kernel(h_hbm, src_hbm, dst_hbm, out_hbm,
                swa, swb, dwa, dwb, rows0, rows1, rows2, acc_sh,
                gs0, gs1, gs2, ssa, ssb, dsa, dsb):
    cid = lax.axis_index("c")
    sid = lax.axis_index("s")
    wid = _worker_id()

    _fill(rows0, KE, 0.0)
    _zero_acc_slice(rows0, acc_sh, sid)

    def _win_start(arr, w, nch, buf, sem):
        pltpu.async_copy(
            arr.at[wid, pl.ds(w * WPB, nch)], buf.at[pl.ds(0, nch)], sem
        )

    def _win_wait(arr, w, nch, buf, sem):
        pltpu.make_async_copy(
            arr.at[wid, pl.ds(w * WPB, nch)], buf.at[pl.ds(0, nch)], sem
        ).wait()

    pltpu.sync_copy(src_hbm.at[wid, pl.ds(0, WPB)], swa)
    pltpu.sync_copy(dst_hbm.at[wid, pl.ds(0, WPB)], dwa)
    _win_start(src_hbm, 1, WPB, swb, ssb)
    _win_start(dst_hbm, 1, WPB, dwb, dsb)
    plsc.subcore_barrier()

    bufs = (rows0, rows1, rows2)
    sems = (gs0, gs1, gs2)

    def _chunk_step(base, j, cur_sw, nxt_sw, cur_dw):
        buf, sem = bufs[j % NBUF], sems[j % NBUF]
        pltpu.make_async_copy(h_hbm.at[cur_sw.at[j]], buf, sem).wait()
        pltpu.sync_copy(buf, acc_sh.at[cur_dw.at[j]], add=True)
        nj = j + NBUF
        tgt_sw, tgt_row = (cur_sw, nj) if nj < WPB else (nxt_sw, nj - WPB)
        if isinstance(base, int):
            if base + nj < NCHP:
                pltpu.async_copy(h_hbm.at[tgt_sw.at[tgt_row]], buf, sem)
        else:
            pltpu.async_copy(h_hbm.at[tgt_sw.at[tgt_row]], buf, sem)

    for b in range(NBUF):
        pltpu.async_copy(h_hbm.at[swa.at[b]], bufs[b], sems[b])

    @pl.loop(0, ((FW - 1) // 2) * 2, step=2)
    def _(w):
        base = w * WPB
        _win_wait(src_hbm, w + 1, WPB, swb, ssb)
        for j in range(WPB):
            _chunk_step(base, j, swa, swb, dwa)

        @pl.when(w + 2 < FW)
        def _():
            _win_start(src_hbm, w + 2, WPB, swa, ssa)
            _win_start(dst_hbm, w + 2, WPB, dwa, dsa)

        _win_wait(dst_hbm, w + 1, WPB, dwb, dsb)
        for j in range(WPB - NBUF):
            _chunk_step(base + WPB, j, swb, swa, dwb)

        @pl.when(w + 2 < FW)
        def _():
            _win_wait(src_hbm, w + 2, WPB, swa, ssa)

        for j in range(WPB - NBUF, WPB):
            _chunk_step(base + WPB, j, swb, swa, dwb)

        @pl.when(w + 3 < FW)
        def _():
            _win_start(src_hbm, w + 3, WPB, swb, ssb)
            _win_start(dst_hbm, w + 3, WPB, dwb, dsb)

        @pl.when(w + 2 < FW)
        def _():
            _win_wait(dst_hbm, w + 2, WPB, dwa, dsa)

    _win_start(src_hbm, FW, TWC, swb, ssb)
    _win_start(dst_hbm, FW, TWC, dwb, dsb)
    base_t = (FW - 1) * WPB
    for j in range(WPB - NBUF):
        _chunk_step(base_t, j, swa, swb, dwa)
    _win_wait(src_hbm, FW, TWC, swb, ssb)
    for j in range(WPB - NBUF, WPB):
        _chunk_step(base_t, j, swa, swb, dwa)
    _win_wait(dst_hbm, FW, TWC, dwb, dsb)
    for j in range(TWC):
        _chunk_step(FW * WPB, j, swb, swb, dwb)

    plsc.subcore_barrier()
    pltpu.sync_copy(
        acc_sh.at[pl.ds(sid * RPS, RPS)],
        out_hbm.at[cid, sid],
    )


_RB = 400
_GRID = N // _RB

_row_spec = pl.BlockSpec((_RB, D), lambda i: (i, 0))
_pair_spec = pl.BlockSpec((NC, _RB, D), lambda i: (0, i, 0))
_w_spec = pl.BlockSpec((D, D), lambda i: (0, 0))
_b_spec = pl.BlockSpec((1, D), lambda i: (0, 0))


def _prep_body(part_ref, x_ref, w_ref, hp_ref, db_ref):
    deg_flat = jnp.sum(part_ref[...], axis=0) + 1.0
    dinv_flat = lax.rsqrt(deg_flat)
    eye = (
        lax.broadcasted_iota(jnp.int32, (D, D), 0)
        == lax.broadcasted_iota(jnp.int32, (D, D), 1)
    ).astype(jnp.float32)
    h = jnp.dot(x_ref[...], w_ref[...], preferred_element_type=jnp.float32)
    for k in range((N + D - 1) // D):
        sz = min(D, N - k * D)
        row = dinv_flat[k : k + 1, :]
        col = jnp.sum(
            jnp.broadcast_to(row, (D, D)) * eye, axis=1, keepdims=True
        )
        blk = jnp.broadcast_to(col, (D, D))[:sz]
        db_ref[pl.ds(k * D, sz), :] = blk
        hp_ref[pl.ds(k * D, sz), :] = h[k * D : k * D + sz] * blk


_prep = pl.pallas_call(
    _prep_body,
    out_shape=[
        jax.ShapeDtypeStruct((N, D), jnp.float32),
        jax.ShapeDtypeStruct((N, D), jnp.float32),
    ],
)


def _mid_body(p_ref, hp_ref, b_ref, w_ref, db_ref, o_ref):
    dinv = db_ref[...]
    pre = dinv * (p_ref[0] + p_ref[1] + hp_ref[...]) + b_ref[...]
    act = jnp.maximum(pre, 0.0)
    h = jnp.dot(act, w_ref[...], preferred_element_type=jnp.float32)
    o_ref[...] = h * dinv


_mid = pl.pallas_call(
    _mid_body,
    grid=(_GRID,),
    in_specs=[_pair_spec, _row_spec, _b_spec, _w_spec, _row_spec],
    out_specs=_row_spec,
    out_shape=jax.ShapeDtypeStruct((N, D), jnp.float32),
)


def _final_body(p_ref, hp_ref, b_ref, db_ref, o_ref):
    o_ref[...] = (
        db_ref[...] * (p_ref[0] + p_ref[1] + hp_ref[...]) + b_ref[...]
    )


_final = pl.pallas_call(
    _final_body,
    grid=(_GRID,),
    in_specs=[_pair_spec, _row_spec, _b_spec, _row_spec],
    out_specs=_row_spec,
    out_shape=jax.ShapeDtypeStruct((N, D), jnp.float32),
)


def kernel(x, edge_index, W1, b1, W2, b2):
    src = edge_index[0].astype(jnp.int32).reshape(NW, NCHP, KE)
    dst = edge_index[1].astype(jnp.int32).reshape(NW, NCHP, KE)
    b1r = b1.reshape(1, D)
    b2r = b2.reshape(1, D)

    hist = _hist_kernel(dst).reshape(NW, HRD // D, D)

    h1p, dinvb = _prep(hist, x, W1)
    p = _agg_kernel(h1p, src, dst).reshape(NC, N, D)
    h2p = _mid(p, h1p, b1r, W2, dinvb)
    q = _agg_kernel(h2p, src, dst).reshape(NC, N, D)
    out = _final(q, h2p, b2r, dinvb)
    return out


# speedup vs baseline: 1.8058x; 1.0083x over previous
import dataclasses
import functools

import jax
import jax.numpy as jnp
from jax import lax
from jax.experimental import pallas as pl
from jax.experimental.pallas import tpu as pltpu
from jax.experimental.pallas import tpu_sc as plsc

N = 10000
E = 320000
D = 128

NC = 2
NS = 16
NW = NC * NS
KE = 80
NCHP = 125
EPW = NCHP * KE
WPB = 8
FW = NCHP // WPB
TWC = NCHP % WPB
NBUF = 4
RPS = N // NS

_mesh = plsc.VectorSubcoreMesh(
    core_axis_name="c", subcore_axis_name="s", num_cores=NC, num_subcores=NS
)


def _worker_id():
    return lax.axis_index("s") * NC + lax.axis_index("c")


def _fill(buf, nrows, value):
    @pl.loop(0, nrows)
    def _(r):
        @pl.loop(0, D // 16)
        def _(cc):
            buf[r, pl.ds(cc * 16, 16)] = jnp.full((16,), value, jnp.float32)


def _zero_acc_slice(zbuf, acc_sh, sid):
    @pl.loop(0, RPS // KE)
    def _(j):
        pltpu.sync_copy(zbuf, acc_sh.at[pl.ds(sid * RPS + j * KE, KE)])

    rem = RPS - (RPS // KE) * KE
    if rem:
        pltpu.sync_copy(
            zbuf.at[pl.ds(0, rem)],
            acc_sh.at[pl.ds(sid * RPS + (RPS // KE) * KE, rem)],
        )


HRD = 10240

_cp = pltpu.CompilerParams()
if "needs_layout_passes" in pltpu.CompilerParams.__dataclass_fields__:
    _cp = dataclasses.replace(_cp, needs_layout_passes=False)


@functools.partial(
    pl.kernel,
    out_type=jax.ShapeDtypeStruct((NW * HRD,), jnp.float32),
    mesh=_mesh,
    compiler_params=_cp,
    scratch_types=[
        pltpu.VMEM((NCHP, KE), jnp.int32),
        pltpu.VMEM((HRD,), jnp.float32),
        pltpu.SemaphoreType.DMA,
    ],
)
def _hist_kernel(dst_hbm, out_hbm, dst_v, lh_v, sem):
    wid = _worker_id()

    pltpu.async_copy(dst_hbm.at[wid], dst_v, sem)

    @pl.loop(0, HRD // 16)
    def _(k):
        lh_v[pl.ds(k * 16, 16)] = jnp.zeros((16,), jnp.float32)

    pltpu.make_async_copy(dst_hbm.at[wid], dst_v, sem).wait()

    ones16 = jnp.full((16,), 1.0, jnp.float32)

    @pl.loop(0, NCHP)
    def _(i):
        @pl.loop(0, KE // 16)
        def _(c):
            idx = dst_v[i, pl.ds(c * 16, 16)]
            plsc.addupdate_scatter(lh_v, [idx], ones16)

    pltpu.sync_copy(lh_v, out_hbm.at[pl.ds(wid * HRD, HRD)])


@functools.partial(
    pl.kernel,
    out_type=jax.ShapeDtypeStruct((NC, NS, RPS, D), jnp.float32),
    mesh=_mesh,
    scratch_types=[
        pltpu.VMEM((WPB, KE), jnp.int32),
        pltpu.VMEM((WPB, KE), jnp.int32),
        pltpu.VMEM((WPB, KE), jnp.int32),
        pltpu.VMEM((WPB, KE), jnp.int32),
        pltpu.VMEM((KE, D), jnp.float32),
        pltpu.VMEM((KE, D), jnp.float32),
        pltpu.VMEM((KE, D), jnp.float32),
        pltpu.VMEM((KE, D), jnp.float32),
        pltpu.VMEM_SHARED((N, D), jnp.float32),
        pltpu.SemaphoreType.DMA,
        pltpu.SemaphoreType.DMA,
        pltpu.SemaphoreType.DMA,
        pltpu.SemaphoreType.DMA,
        pltpu.SemaphoreType.DMA,
        pltpu.SemaphoreType.DMA,
        pltpu.SemaphoreType.DMA,
        pltpu.SemaphoreType.DMA,
    ],
)
def _agg_kernel(h_hbm, src_hbm, dst_hbm, out_hbm,
                swa, swb, dwa, dwb, rows0, rows1, rows2, rows3, acc_sh,
                gs0, gs1, gs2, gs3, ssa, ssb, dsa, dsb):
    cid = lax.axis_index("c")
    sid = lax.axis_index("s")
    wid = _worker_id()

    _fill(rows0, KE, 0.0)
    _zero_acc_slice(rows0, acc_sh, sid)

    def _win_start(arr, w, nch, buf, sem):
        pltpu.async_copy(
            arr.at[wid, pl.ds(w * WPB, nch)], buf.at[pl.ds(0, nch)], sem
        )

    def _win_wait(arr, w, nch, buf, sem):
        pltpu.make_async_copy(
            arr.at[wid, pl.ds(w * WPB, nch)], buf.at[pl.ds(0, nch)], sem
        ).wait()

    pltpu.sync_copy(src_hbm.at[wid, pl.ds(0, WPB)], swa)
    pltpu.sync_copy(dst_hbm.at[wid, pl.ds(0, WPB)], dwa)
    _win_start(src_hbm, 1, WPB, swb, ssb)
    _win_start(dst_hbm, 1, WPB, dwb, dsb)
    plsc.subcore_barrier()

    bufs = (rows0, rows1, rows2, rows3)
    sems = (gs0, gs1, gs2, gs3)

    def _chunk_step(base, j, cur_sw, nxt_sw, cur_dw):
        buf, sem = bufs[j % NBUF], sems[j % NBUF]
        pltpu.make_async_copy(h_hbm.at[cur_sw.at[j]], buf, sem).wait()
        pltpu.sync_copy(buf, acc_sh.at[cur_dw.at[j]], add=True)
        nj = j + NBUF
        tgt_sw, tgt_row = (cur_sw, nj) if nj < WPB else (nxt_sw, nj - WPB)
        if isinstance(base, int):
            if base + nj < NCHP:
                pltpu.async_copy(h_hbm.at[tgt_sw.at[tgt_row]], buf, sem)
        else:
            pltpu.async_copy(h_hbm.at[tgt_sw.at[tgt_row]], buf, sem)

    for b in range(NBUF):
        pltpu.async_copy(h_hbm.at[swa.at[b]], bufs[b], sems[b])

    @pl.loop(0, ((FW - 1) // 2) * 2, step=2)
    def _(w):
        base = w * WPB
        _win_wait(src_hbm, w + 1, WPB, swb, ssb)
        for j in range(WPB):
            _chunk_step(base, j, swa, swb, dwa)

        @pl.when(w + 2 < FW)
        def _():
            _win_start(src_hbm, w + 2, WPB, swa, ssa)
            _win_start(dst_hbm, w + 2, WPB, dwa, dsa)

        _win_wait(dst_hbm, w + 1, WPB, dwb, dsb)
        for j in range(WPB - NBUF):
            _chunk_step(base + WPB, j, swb, swa, dwb)

        @pl.when(w + 2 < FW)
        def _():
            _win_wait(src_hbm, w + 2, WPB, swa, ssa)

        for j in range(WPB - NBUF, WPB):
            _chunk_step(base + WPB, j, swb, swa, dwb)

        @pl.when(w + 3 < FW)
        def _():
            _win_start(src_hbm, w + 3, WPB, swb, ssb)
            _win_start(dst_hbm, w + 3, WPB, dwb, dsb)

        @pl.when(w + 2 < FW)
        def _():
            _win_wait(dst_hbm, w + 2, WPB, dwa, dsa)

    _win_start(src_hbm, FW, TWC, swb, ssb)
    _win_start(dst_hbm, FW, TWC, dwb, dsb)
    base_t = (FW - 1) * WPB
    for j in range(WPB - NBUF):
        _chunk_step(base_t, j, swa, swb, dwa)
    _win_wait(src_hbm, FW, TWC, swb, ssb)
    for j in range(WPB - NBUF, WPB):
        _chunk_step(base_t, j, swa, swb, dwa)
    _win_wait(dst_hbm, FW, TWC, dwb, dsb)
    for j in range(TWC):
        _chunk_step(FW * WPB, j, swb, swb, dwb)

    plsc.subcore_barrier()
    pltpu.sync_copy(
        acc_sh.at[pl.ds(sid * RPS, RPS)],
        out_hbm.at[cid, sid],
    )


_RB = 400
_GRID = N // _RB

_row_spec = pl.BlockSpec((_RB, D), lambda i: (i, 0))
_pair_spec = pl.BlockSpec((NC, _RB, D), lambda i: (0, i, 0))
_w_spec = pl.BlockSpec((D, D), lambda i: (0, 0))
_b_spec = pl.BlockSpec((1, D), lambda i: (0, 0))


def _prep_body(part_ref, x_ref, w_ref, hp_ref, db_ref):
    deg_flat = jnp.sum(part_ref[...], axis=0) + 1.0
    dinv_flat = lax.rsqrt(deg_flat)
    eye = (
        lax.broadcasted_iota(jnp.int32, (D, D), 0)
        == lax.broadcasted_iota(jnp.int32, (D, D), 1)
    ).astype(jnp.float32)
    h = jnp.dot(x_ref[...], w_ref[...], preferred_element_type=jnp.float32)
    for k in range((N + D - 1) // D):
        sz = min(D, N - k * D)
        row = dinv_flat[k : k + 1, :]
        col = jnp.sum(
            jnp.broadcast_to(row, (D, D)) * eye, axis=1, keepdims=True
        )
        blk = jnp.broadcast_to(col, (D, D))[:sz]
        db_ref[pl.ds(k * D, sz), :] = blk
        hp_ref[pl.ds(k * D, sz), :] = h[k * D : k * D + sz] * blk


_prep = pl.pallas_call(
    _prep_body,
    out_shape=[
        jax.ShapeDtypeStruct((N, D), jnp.float32),
        jax.ShapeDtypeStruct((N, D), jnp.float32),
    ],
)


def _mid_body(p_ref, hp_ref, b_ref, w_ref, db_ref, o_ref):
    dinv = db_ref[...]
    pre = dinv * (p_ref[0] + p_ref[1] + hp_ref[...]) + b_ref[...]
    act = jnp.maximum(pre, 0.0)
    h = jnp.dot(act, w_ref[...], preferred_element_type=jnp.float32)
    o_ref[...] = h * dinv


_mid = pl.pallas_call(
    _mid_body,
    grid=(_GRID,),
    in_specs=[_pair_spec, _row_spec, _b_spec, _w_spec, _row_spec],
    out_specs=_row_spec,
    out_shape=jax.ShapeDtypeStruct((N, D), jnp.float32),
)


def _final_body(p_ref, hp_ref, b_ref, db_ref, o_ref):
    o_ref[...] = (
        db_ref[...] * (p_ref[0] + p_ref[1] + hp_ref[...]) + b_ref[...]
    )


_final = pl.pallas_call(
    _final_body,
    grid=(_GRID,),
    in_specs=[_pair_spec, _row_spec, _b_spec, _row_spec],
    out_specs=_row_spec,
    out_shape=jax.ShapeDtypeStruct((N, D), jnp.float32),
)


def kernel(x, edge_index, W1, b1, W2, b2):
    src = edge_index[0].astype(jnp.int32).reshape(NW, NCHP, KE)
    dst = edge_index[1].astype(jnp.int32).reshape(NW, NCHP, KE)
    b1r = b1.reshape(1, D)
    b2r = b2.reshape(1, D)

    hist = _hist_kernel(dst).reshape(NW, HRD // D, D)

    h1p, dinvb = _prep(hist, x, W1)
    p = _agg_kernel(h1p, src, dst).reshape(NC, N, D)
    h2p = _mid(p, h1p, b1r, W2, dinvb)
    q = _agg_kernel(h2p, src, dst).reshape(NC, N, D)
    out = _final(q, h2p, b2r, dinvb)
    return out
